# baseline (speedup 1.0000x reference)
import functools
import jax
import jax.numpy as jnp
from jax import lax
from jax.experimental import pallas as pl
from jax.experimental.pallas import tpu as pltpu
from jax.experimental.pallas import tpu_sc as plsc

N_NODES = 10000
N_EDGES = 320000
IN_DIM = 128
HID = 256
OUT = 128

NC = 2
NS = 16
CHUNK = 64
EPT = N_EDGES // NS
NCHUNK = -(-EPT // CHUNK)
EPAD = NCHUNK * CHUNK - EPT
PPB = NS * NCHUNK
ACC_ROWS = 10016
RPT = 624
TAIL = N_NODES - NS * RPT
ZTAIL = ACC_ROWS - NS * RPT
HALF = 5120


def _seg_body(n_phases):
    def body(table, pairs, z128, out, idx_v, rows_v, sems, acc, sem):
        c = lax.axis_index("c")
        s = lax.axis_index("s")
        rbase = pl.multiple_of(s * RPT, 8)
        for p in range(n_phases):
            blk = c * n_phases + p
            pltpu.sync_copy(z128, acc.at[pl.ds(rbase, RPT)])

            @pl.when(s == 0)
            def _():
                pltpu.sync_copy(z128.at[pl.ds(0, ZTAIL)],
                                acc.at[pl.ds(NS * RPT, ZTAIL)])

            plsc.subcore_barrier()
            pbase = blk * PPB + s * NCHUNK

            def load_and_fire(t, b):
                pltpu.sync_copy(pairs.at[pbase + t], idx_v.at[b])
                pltpu.async_copy(table.at[idx_v.at[b, 0]], rows_v.at[b],
                                 sems.at[b])

            load_and_fire(0, 0)

            def chunk(t, carry):
                b = lax.rem(t, 2)

                @pl.when(t + 1 < NCHUNK)
                def _():
                    load_and_fire(t + 1, 1 - b)

                pltpu.make_async_copy(table.at[idx_v.at[b, 0]], rows_v.at[b],
                                      sems.at[b]).wait()
                pltpu.sync_copy(rows_v.at[b], acc.at[idx_v.at[b, 1]], add=True)
                return carry

            lax.fori_loop(0, NCHUNK, chunk, 0)
            plsc.subcore_barrier()
            obase = pl.multiple_of(blk * N_NODES + rbase, 8)
            pltpu.sync_copy(acc.at[pl.ds(rbase, RPT)],
                            out.at[pl.ds(obase, RPT)])

            @pl.when(s == 0)
            def _():
                tb = pl.multiple_of(blk * N_NODES + NS * RPT, 8)
                pltpu.sync_copy(acc.at[pl.ds(NS * RPT, TAIL)],
                                out.at[pl.ds(tb, TAIL)])

            plsc.subcore_barrier()

    return body


def _deg_body(pairs, z3, identh, dout, idx_v, ldeg, idv, dshared, sem):
    c = lax.axis_index("c")
    s = lax.axis_index("s")
    pltpu.sync_copy(identh, idv)

    @pl.when(s == 0)
    def _():
        pltpu.sync_copy(z3.at[0], dshared.at[pl.ds(0, 40)])
        pltpu.sync_copy(z3.at[1], dshared.at[pl.ds(40, 40)])

    plsc.subcore_barrier()
    pbase = c * PPB + s * NCHUNK
    iota16 = lax.iota(jnp.int32, 16)
    for half in range(2):
        lo = half * HALF
        pltpu.sync_copy(z3, ldeg)

        def chunk(t, carry):
            pltpu.sync_copy(pairs.at[pbase + t], idx_v)
            for q in range(CHUNK // 16):
                v = idx_v[1, pl.ds(q * 16, 16)]
                vv = v - lo
                m = jnp.logical_and(vv >= 0, vv < HALF)
                vc = jnp.clip(vv, 0, HALF - 1)
                hi = lax.shift_right_logical(vc, 7)
                lo7 = lax.bitwise_and(vc, 127)
                cur = plsc.load_gather(ldeg, [iota16, hi, lo7], mask=m)
                plsc.store_scatter(ldeg, [iota16, hi, lo7], cur + 1.0, mask=m)
            return carry

        lax.fori_loop(0, NCHUNK, chunk, 0)
        for k2 in range(16):
            pltpu.sync_copy(ldeg.at[k2], dshared.at[idv.at[half]], add=True)
    plsc.subcore_barrier()

    @pl.when(s == 0)
    def _():
        pltpu.sync_copy(dshared, dout.at[pl.ds(c * 80, 80)])


@functools.lru_cache(maxsize=1)
def _sc_kernels():
    mesh = plsc.VectorSubcoreMesh(core_axis_name="c", subcore_axis_name="s")
    seg_scratch = [
        pltpu.VMEM((2, 2, CHUNK), jnp.int32),
        pltpu.VMEM((2, CHUNK, IN_DIM), jnp.float32),
        pltpu.SemaphoreType.DMA((2,)),
        pltpu.VMEM_SHARED((ACC_ROWS, IN_DIM), jnp.float32),
        pltpu.SemaphoreType.DMA,
    ]
    seg1 = pl.kernel(
        _seg_body(1),
        out_type=jax.ShapeDtypeStruct((NC * N_NODES, IN_DIM), jnp.float32),
        mesh=mesh,
        scratch_types=list(seg_scratch),
    )
    seg2 = pl.kernel(
        _seg_body(2),
        out_type=jax.ShapeDtypeStruct((2 * NC * N_NODES, IN_DIM), jnp.float32),
        mesh=mesh,
        scratch_types=list(seg_scratch),
    )
    deg = pl.kernel(
        _deg_body,
        out_type=jax.ShapeDtypeStruct((NC * 80, 128), jnp.float32),
        compiler_params=pltpu.CompilerParams(needs_layout_passes=False),
        mesh=mesh,
        scratch_types=[
            pltpu.VMEM((2, CHUNK), jnp.int32),
            pltpu.VMEM((16, 40, 128), jnp.float32),
            pltpu.VMEM((2, 40), jnp.int32),
            pltpu.VMEM_SHARED((80, 128), jnp.float32),
            pltpu.SemaphoreType.DMA,
        ],
    )
    return seg1, seg2, deg


R = 400
NB = N_NODES // R


def _tc1_body(feats_ref, s1_ref, deg_ref, ws_ref, wn_ref, b_ref, y_ref):
    d = jnp.maximum(deg_ref[...], 1.0)
    hn = s1_ref[...] / d
    y = (jnp.dot(feats_ref[...], ws_ref[0], preferred_element_type=jnp.float32)
         + jnp.dot(hn, wn_ref[0], preferred_element_type=jnp.float32)
         + b_ref[0, 0:1, :])
    y_ref[...] = jnp.maximum(y, 0.0)


def _tc1(feats, s1, deg, ws, wn, b):
    return pl.pallas_call(
        _tc1_body,
        grid=(NB, 4),
        in_specs=[
            pl.BlockSpec((R, IN_DIM), lambda i, j: (i, 0)),
            pl.BlockSpec((R, IN_DIM), lambda i, j: ((j // 2) * NB + i, 0)),
            pl.BlockSpec((R, 1), lambda i, j: ((j // 2) * NB + i, 0)),
            pl.BlockSpec((1, IN_DIM, IN_DIM), lambda i, j: (j // 2, 0, j % 2)),
            pl.BlockSpec((1, IN_DIM, IN_DIM), lambda i, j: (j // 2, 0, j % 2)),
            pl.BlockSpec((1, 8, IN_DIM), lambda i, j: (j // 2, 0, j % 2)),
        ],
        out_specs=pl.BlockSpec((R, IN_DIM), lambda i, j: (j * NB + i, 0)),
        out_shape=jax.ShapeDtypeStruct((2 * NC * N_NODES, IN_DIM), jnp.float32),
    )(feats, s1, deg, ws, wn, b)


def _tc2_body(yl_ref, yr_ref, s2l_ref, s2r_ref, deg_ref,
              ws_ref, wn_ref, b_ref, wl_ref, out_ref, acc_ref):
    bidx = pl.program_id(0)
    i = pl.program_id(1)
    h1 = jnp.concatenate([yl_ref[...], yr_ref[...]], axis=1)
    d = jnp.maximum(deg_ref[...], 1.0)
    hn = jnp.concatenate([s2l_ref[...], s2r_ref[...]], axis=1) / d
    h2 = (jnp.dot(h1, ws_ref[0], preferred_element_type=jnp.float32)
          + jnp.dot(hn, wn_ref[0], preferred_element_type=jnp.float32)
          + b_ref[0, 0:1, :])
    h2 = jnp.maximum(h2, 0.0)
    colsum = jnp.sum(h2, axis=0, keepdims=True)

    @pl.when(i == 0)
    def _():
        acc_ref[pl.ds(bidx, 1), :] = colsum

    @pl.when(i > 0)
    def _():
        acc_ref[pl.ds(bidx, 1), :] = acc_ref[pl.ds(bidx, 1), :] + colsum

    @pl.when(jnp.logical_and(bidx == 1, i == NB - 1))
    def _():
        inv = 1.0 / N_NODES
        ra = jnp.dot(acc_ref[0:1, :] * inv, wl_ref[0],
                     preferred_element_type=jnp.float32)
        ra = jnp.maximum(ra, 0.0)
        rb = jnp.dot(acc_ref[1:2, :] * inv, wl_ref[1],
                     preferred_element_type=jnp.float32)
        out_ref[...] = jax.nn.sigmoid((ra + rb) * 0.5)


def _tc2(y, s2, deg, ws, wn, b, wl):
    return pl.pallas_call(
        _tc2_body,
        grid=(2, NB),
        in_specs=[
            pl.BlockSpec((R, IN_DIM), lambda bi, i: ((bi * 2) * NB + i, 0)),
            pl.BlockSpec((R, IN_DIM), lambda bi, i: ((bi * 2 + 1) * NB + i, 0)),
            pl.BlockSpec((R, IN_DIM), lambda bi, i: ((bi * 2) * NB + i, 0)),
            pl.BlockSpec((R, IN_DIM), lambda bi, i: ((bi * 2 + 1) * NB + i, 0)),
            pl.BlockSpec((R, 1), lambda bi, i: (bi * NB + i, 0)),
            pl.BlockSpec((1, HID, HID), lambda bi, i: (bi, 0, 0)),
            pl.BlockSpec((1, HID, HID), lambda bi, i: (bi, 0, 0)),
            pl.BlockSpec((1, 8, HID), lambda bi, i: (bi, 0, 0)),
            pl.BlockSpec((2, HID, OUT), lambda bi, i: (0, 0, 0)),
        ],
        out_specs=pl.BlockSpec((1, OUT), lambda bi, i: (0, 0)),
        out_shape=jax.ShapeDtypeStruct((1, OUT), jnp.float32),
        scratch_shapes=[pltpu.VMEM((8, HID), jnp.float32)],
    )(y, y, s2, s2, deg, ws, wn, b, wl)


def kernel(feats, edge_index1, edge_index2,
           W1_self_a, W1_neigh_a, b1_a, W2_self_a, W2_neigh_a, b2_a, Wl_a,
           W1_self_b, W1_neigh_b, b1_b, W2_self_b, W2_neigh_b, b2_b, Wl_b):
    e1 = edge_index1.astype(jnp.int32)
    e2 = edge_index2.astype(jnp.int32)

    def pack(src, dst, srcoff):
        s = jnp.pad((src + srcoff).reshape(NS, EPT), ((0, 0), (0, EPAD)))
        d = jnp.pad(dst.reshape(NS, EPT), ((0, 0), (0, EPAD)),
                    constant_values=N_NODES)
        s = s.reshape(NS, NCHUNK, 1, CHUNK)
        d = d.reshape(NS, NCHUNK, 1, CHUNK)
        return jnp.concatenate([s, d], axis=2).reshape(PPB, 2, CHUNK)

    pairs1 = jnp.concatenate([pack(e1[0], e1[1], 0),
                              pack(e2[0], e2[1], 0)])
    pairs2 = jnp.concatenate([pack(e1[0], e1[1], 0),
                              pack(e1[0], e1[1], N_NODES),
                              pack(e2[0], e2[1], 2 * N_NODES),
                              pack(e2[0], e2[1], 3 * N_NODES)])
    z128 = jnp.zeros((RPT, IN_DIM), jnp.float32)
    z3 = jnp.zeros((16, 40, 128), jnp.float32)
    ident2 = jnp.arange(2 * 40, dtype=jnp.int32).reshape(2, 40)

    sc_seg1, sc_seg2, sc_deg = _sc_kernels()
    s1 = sc_seg1(feats, pairs1, z128)
    deg_raw = sc_deg(pairs1, z3, ident2)
    deg = deg_raw.reshape(NC, 80 * 128)[:, :N_NODES].reshape(NC * N_NODES, 1)

    ws1 = jnp.stack([W1_self_a, W1_self_b])
    wn1 = jnp.stack([W1_neigh_a, W1_neigh_b])
    b1 = jnp.tile(jnp.stack([b1_a, b1_b])[:, None, :], (1, 8, 1))
    y = _tc1(feats, s1, deg, ws1, wn1, b1)

    s2 = sc_seg2(y, pairs2, z128)

    ws2 = jnp.stack([W2_self_a, W2_self_b])
    wn2 = jnp.stack([W2_neigh_a, W2_neigh_b])
    b2 = jnp.tile(jnp.stack([b2_a, b2_b])[:, None, :], (1, 8, 1))
    wl = jnp.stack([Wl_a, Wl_b])
    return _tc2(y, s2, deg, ws2, wn2, b2, wl)


# speedup vs baseline: 1.2028x; 1.2028x over previous
import functools
import jax
import jax.numpy as jnp
from jax import lax
from jax.experimental import pallas as pl
from jax.experimental.pallas import tpu as pltpu
from jax.experimental.pallas import tpu_sc as plsc

N_NODES = 10000
N_EDGES = 320000
IN_DIM = 128
HID = 256
OUT = 128

NC = 2
NS = 16
CHUNK = 80
EPT = N_EDGES // NS
NCHUNK = -(-EPT // CHUNK)
EPAD = NCHUNK * CHUNK - EPT
PPB = NS * NCHUNK
ACC_ROWS = 10016
RPT = 624
TAIL = N_NODES - NS * RPT
ZTAIL = ACC_ROWS - NS * RPT
HALF = 5120


def _seg_body(n_phases):
    def body(table, pairs, z128, out, idx_v, rows_v, sems, acc, sem):
        c = lax.axis_index("c")
        s = lax.axis_index("s")
        rbase = pl.multiple_of(s * RPT, 8)
        for p in range(n_phases):
            blk = c * n_phases + p
            pltpu.sync_copy(z128, acc.at[pl.ds(rbase, RPT)])

            @pl.when(s == 0)
            def _():
                pltpu.sync_copy(z128.at[pl.ds(0, ZTAIL)],
                                acc.at[pl.ds(NS * RPT, ZTAIL)])

            plsc.subcore_barrier()
            pbase = blk * PPB + s * NCHUNK

            def load_and_fire(t, b):
                pltpu.sync_copy(pairs.at[pbase + t], idx_v.at[b])
                pltpu.async_copy(table.at[idx_v.at[b, 0]], rows_v.at[b],
                                 sems.at[b])

            load_and_fire(0, 0)

            def chunk(t, carry):
                b = lax.rem(t, 2)

                @pl.when(t + 1 < NCHUNK)
                def _():
                    load_and_fire(t + 1, 1 - b)

                pltpu.make_async_copy(table.at[idx_v.at[b, 0]], rows_v.at[b],
                                      sems.at[b]).wait()
                pltpu.sync_copy(rows_v.at[b], acc.at[idx_v.at[b, 1]], add=True)
                return carry

            lax.fori_loop(0, NCHUNK, chunk, 0)
            plsc.subcore_barrier()
            obase = pl.multiple_of(blk * N_NODES + rbase, 8)
            pltpu.sync_copy(acc.at[pl.ds(rbase, RPT)],
                            out.at[pl.ds(obase, RPT)])

            @pl.when(s == 0)
            def _():
                tb = pl.multiple_of(blk * N_NODES + NS * RPT, 8)
                pltpu.sync_copy(acc.at[pl.ds(NS * RPT, TAIL)],
                                out.at[pl.ds(tb, TAIL)])

            plsc.subcore_barrier()

    return body


def _deg_body(pairs, z3, identh, dout, idx_v, ldeg, idv, dshared, sem):
    c = lax.axis_index("c")
    s = lax.axis_index("s")
    pltpu.sync_copy(identh, idv)

    @pl.when(s == 0)
    def _():
        pltpu.sync_copy(z3.at[0], dshared.at[pl.ds(0, 40)])
        pltpu.sync_copy(z3.at[1], dshared.at[pl.ds(40, 40)])

    plsc.subcore_barrier()
    pbase = c * PPB + s * NCHUNK
    iota16 = lax.iota(jnp.int32, 16)
    for half in range(2):
        lo = half * HALF
        pltpu.sync_copy(z3, ldeg)

        def chunk(t, carry):
            pltpu.sync_copy(pairs.at[pbase + t], idx_v)
            for q in range(CHUNK // 16):
                v = idx_v[1, pl.ds(q * 16, 16)]
                vv = v - lo
                m = jnp.logical_and(vv >= 0, vv < HALF)
                vc = jnp.clip(vv, 0, HALF - 1)
                hi = lax.shift_right_logical(vc, 7)
                lo7 = lax.bitwise_and(vc, 127)
                cur = plsc.load_gather(ldeg, [iota16, hi, lo7], mask=m)
                plsc.store_scatter(ldeg, [iota16, hi, lo7], cur + 1.0, mask=m)
            return carry

        lax.fori_loop(0, NCHUNK, chunk, 0)
        for k2 in range(16):
            pltpu.sync_copy(ldeg.at[k2], dshared.at[idv.at[half]], add=True)
    plsc.subcore_barrier()

    @pl.when(s == 0)
    def _():
        pltpu.sync_copy(dshared, dout.at[pl.ds(c * 80, 80)])


@functools.lru_cache(maxsize=1)
def _sc_kernels():
    mesh = plsc.VectorSubcoreMesh(core_axis_name="c", subcore_axis_name="s")
    seg_scratch = [
        pltpu.VMEM((2, 2, CHUNK), jnp.int32),
        pltpu.VMEM((2, CHUNK, IN_DIM), jnp.float32),
        pltpu.SemaphoreType.DMA((2,)),
        pltpu.VMEM_SHARED((ACC_ROWS, IN_DIM), jnp.float32),
        pltpu.SemaphoreType.DMA,
    ]
    seg1 = pl.kernel(
        _seg_body(1),
        out_type=jax.ShapeDtypeStruct((NC * N_NODES, IN_DIM), jnp.float32),
        mesh=mesh,
        scratch_types=list(seg_scratch),
    )
    seg2 = pl.kernel(
        _seg_body(2),
        out_type=jax.ShapeDtypeStruct((2 * NC * N_NODES, IN_DIM), jnp.float32),
        mesh=mesh,
        scratch_types=list(seg_scratch),
    )
    deg = pl.kernel(
        _deg_body,
        out_type=jax.ShapeDtypeStruct((NC * 80, 128), jnp.float32),
        compiler_params=pltpu.CompilerParams(needs_layout_passes=False),
        mesh=mesh,
        scratch_types=[
            pltpu.VMEM((2, CHUNK), jnp.int32),
            pltpu.VMEM((16, 40, 128), jnp.float32),
            pltpu.VMEM((2, 40), jnp.int32),
            pltpu.VMEM_SHARED((80, 128), jnp.float32),
            pltpu.SemaphoreType.DMA,
        ],
    )
    return seg1, seg2, deg


R = 400
NB = N_NODES // R


def _tc1_body(feats_ref, s1_ref, deg_ref, ws_ref, wn_ref, b_ref, y_ref):
    d = jnp.maximum(deg_ref[...], 1.0)
    hn = s1_ref[...] / d
    y = (jnp.dot(feats_ref[...], ws_ref[0], preferred_element_type=jnp.float32)
         + jnp.dot(hn, wn_ref[0], preferred_element_type=jnp.float32)
         + b_ref[0, 0:1, :])
    y_ref[...] = jnp.maximum(y, 0.0)


def _tc1(feats, s1, deg, ws, wn, b):
    return pl.pallas_call(
        _tc1_body,
        grid=(NB, 4),
        in_specs=[
            pl.BlockSpec((R, IN_DIM), lambda i, j: (i, 0)),
            pl.BlockSpec((R, IN_DIM), lambda i, j: ((j // 2) * NB + i, 0)),
            pl.BlockSpec((R, 1), lambda i, j: ((j // 2) * NB + i, 0)),
            pl.BlockSpec((1, IN_DIM, IN_DIM), lambda i, j: (j // 2, 0, j % 2)),
            pl.BlockSpec((1, IN_DIM, IN_DIM), lambda i, j: (j // 2, 0, j % 2)),
            pl.BlockSpec((1, 8, IN_DIM), lambda i, j: (j // 2, 0, j % 2)),
        ],
        out_specs=pl.BlockSpec((R, IN_DIM), lambda i, j: (j * NB + i, 0)),
        out_shape=jax.ShapeDtypeStruct((2 * NC * N_NODES, IN_DIM), jnp.float32),
    )(feats, s1, deg, ws, wn, b)


def _tc2_body(yl_ref, yr_ref, s2l_ref, s2r_ref, deg_ref,
              ws_ref, wn_ref, b_ref, wl_ref, out_ref, acc_ref):
    bidx = pl.program_id(0)
    i = pl.program_id(1)
    h1 = jnp.concatenate([yl_ref[...], yr_ref[...]], axis=1)
    d = jnp.maximum(deg_ref[...], 1.0)
    hn = jnp.concatenate([s2l_ref[...], s2r_ref[...]], axis=1) / d
    h2 = (jnp.dot(h1, ws_ref[0], preferred_element_type=jnp.float32)
          + jnp.dot(hn, wn_ref[0], preferred_element_type=jnp.float32)
          + b_ref[0, 0:1, :])
    h2 = jnp.maximum(h2, 0.0)
    colsum = jnp.sum(h2, axis=0, keepdims=True)

    @pl.when(i == 0)
    def _():
        acc_ref[pl.ds(bidx, 1), :] = colsum

    @pl.when(i > 0)
    def _():
        acc_ref[pl.ds(bidx, 1), :] = acc_ref[pl.ds(bidx, 1), :] + colsum

    @pl.when(jnp.logical_and(bidx == 1, i == NB - 1))
    def _():
        inv = 1.0 / N_NODES
        ra = jnp.dot(acc_ref[0:1, :] * inv, wl_ref[0],
                     preferred_element_type=jnp.float32)
        ra = jnp.maximum(ra, 0.0)
        rb = jnp.dot(acc_ref[1:2, :] * inv, wl_ref[1],
                     preferred_element_type=jnp.float32)
        out_ref[...] = jax.nn.sigmoid((ra + rb) * 0.5)


def _tc2(y, s2, deg, ws, wn, b, wl):
    return pl.pallas_call(
        _tc2_body,
        grid=(2, NB),
        in_specs=[
            pl.BlockSpec((R, IN_DIM), lambda bi, i: ((bi * 2) * NB + i, 0)),
            pl.BlockSpec((R, IN_DIM), lambda bi, i: ((bi * 2 + 1) * NB + i, 0)),
            pl.BlockSpec((R, IN_DIM), lambda bi, i: ((bi * 2) * NB + i, 0)),
            pl.BlockSpec((R, IN_DIM), lambda bi, i: ((bi * 2 + 1) * NB + i, 0)),
            pl.BlockSpec((R, 1), lambda bi, i: (bi * NB + i, 0)),
            pl.BlockSpec((1, HID, HID), lambda bi, i: (bi, 0, 0)),
            pl.BlockSpec((1, HID, HID), lambda bi, i: (bi, 0, 0)),
            pl.BlockSpec((1, 8, HID), lambda bi, i: (bi, 0, 0)),
            pl.BlockSpec((2, HID, OUT), lambda bi, i: (0, 0, 0)),
        ],
        out_specs=pl.BlockSpec((1, OUT), lambda bi, i: (0, 0)),
        out_shape=jax.ShapeDtypeStruct((1, OUT), jnp.float32),
        scratch_shapes=[pltpu.VMEM((8, HID), jnp.float32)],
    )(y, y, s2, s2, deg, ws, wn, b, wl)


def kernel(feats, edge_index1, edge_index2,
           W1_self_a, W1_neigh_a, b1_a, W2_self_a, W2_neigh_a, b2_a, Wl_a,
           W1_self_b, W1_neigh_b, b1_b, W2_self_b, W2_neigh_b, b2_b, Wl_b):
    e1 = edge_index1.astype(jnp.int32)
    e2 = edge_index2.astype(jnp.int32)

    def pack(src, dst, srcoff):
        s = jnp.pad((src + srcoff).reshape(NS, EPT), ((0, 0), (0, EPAD)))
        d = jnp.pad(dst.reshape(NS, EPT), ((0, 0), (0, EPAD)),
                    constant_values=N_NODES)
        s = s.reshape(NS, NCHUNK, 1, CHUNK)
        d = d.reshape(NS, NCHUNK, 1, CHUNK)
        return jnp.concatenate([s, d], axis=2).reshape(PPB, 2, CHUNK)

    pairs1 = jnp.concatenate([pack(e1[0], e1[1], 0),
                              pack(e2[0], e2[1], 0)])
    pairs2 = jnp.concatenate([pack(e1[0], e1[1], 0),
                              pack(e1[0], e1[1], N_NODES),
                              pack(e2[0], e2[1], 2 * N_NODES),
                              pack(e2[0], e2[1], 3 * N_NODES)])
    z128 = jnp.zeros((RPT, IN_DIM), jnp.float32)
    z3 = jnp.zeros((16, 40, 128), jnp.float32)
    ident2 = jnp.arange(2 * 40, dtype=jnp.int32).reshape(2, 40)

    sc_seg1, sc_seg2, sc_deg = _sc_kernels()
    s1 = sc_seg1(feats, pairs1, z128)
    deg_raw = sc_deg(pairs1, z3, ident2)
    deg = deg_raw.reshape(NC, 80 * 128)[:, :N_NODES].reshape(NC * N_NODES, 1)

    ws1 = jnp.stack([W1_self_a, W1_self_b])
    wn1 = jnp.stack([W1_neigh_a, W1_neigh_b])
    b1 = jnp.tile(jnp.stack([b1_a, b1_b])[:, None, :], (1, 8, 1))
    y = _tc1(feats, s1, deg, ws1, wn1, b1)

    s2 = sc_seg2(y, pairs2, z128)

    ws2 = jnp.stack([W2_self_a, W2_self_b])
    wn2 = jnp.stack([W2_neigh_a, W2_neigh_b])
    b2 = jnp.tile(jnp.stack([b2_a, b2_b])[:, None, :], (1, 8, 1))
    wl = jnp.stack([Wl_a, Wl_b])
    return _tc2(y, s2, deg, ws2, wn2, b2, wl)


# speedup vs baseline: 1.3391x; 1.1134x over previous
import functools
import jax
import jax.numpy as jnp
from jax import lax
from jax.experimental import pallas as pl
from jax.experimental.pallas import tpu as pltpu
from jax.experimental.pallas import tpu_sc as plsc

N_NODES = 10000
N_EDGES = 320000
IN_DIM = 128
HID = 256
OUT = 128

NC = 2
NS = 16
CHUNK = 80
EPT = N_EDGES // NS
NCHUNK = -(-EPT // CHUNK)
EPAD = NCHUNK * CHUNK - EPT
PPB = NS * NCHUNK
ACC_ROWS = 10016
RPT = 624
TAIL = N_NODES - NS * RPT
ZTAIL = ACC_ROWS - NS * RPT
HALF = 5120


def _seg_body(n_phases):
    def body(table, pairs, z128, out, idx_v, rows_v, sems, acc, sem):
        c = lax.axis_index("c")
        s = lax.axis_index("s")
        rbase = pl.multiple_of(s * RPT, 8)
        for p in range(n_phases):
            blk = c * n_phases + p
            pltpu.sync_copy(z128, acc.at[pl.ds(rbase, RPT)])

            @pl.when(s == 0)
            def _():
                pltpu.sync_copy(z128.at[pl.ds(0, ZTAIL)],
                                acc.at[pl.ds(NS * RPT, ZTAIL)])

            plsc.subcore_barrier()
            pbase = blk * PPB + s * NCHUNK

            def load_and_fire(t, b):
                pltpu.sync_copy(pairs.at[pbase + t], idx_v.at[b])
                pltpu.async_copy(table.at[idx_v.at[b, 0]], rows_v.at[b],
                                 sems.at[b])

            load_and_fire(0, 0)

            def chunk(t, carry):
                b = lax.rem(t, 2)

                @pl.when(t + 1 < NCHUNK)
                def _():
                    load_and_fire(t + 1, 1 - b)

                pltpu.make_async_copy(table.at[idx_v.at[b, 0]], rows_v.at[b],
                                      sems.at[b]).wait()
                pltpu.sync_copy(rows_v.at[b], acc.at[idx_v.at[b, 1]], add=True)
                return carry

            lax.fori_loop(0, NCHUNK, chunk, 0)
            plsc.subcore_barrier()
            obase = pl.multiple_of(blk * N_NODES + rbase, 8)
            pltpu.sync_copy(acc.at[pl.ds(rbase, RPT)],
                            out.at[pl.ds(obase, RPT)])

            @pl.when(s == 0)
            def _():
                tb = pl.multiple_of(blk * N_NODES + NS * RPT, 8)
                pltpu.sync_copy(acc.at[pl.ds(NS * RPT, TAIL)],
                                out.at[pl.ds(tb, TAIL)])

            plsc.subcore_barrier()

    return body


def _deg_body(pairs, z3, identh, dout, idx_v, ldeg, idv, sems, dshared, sem):
    c = lax.axis_index("c")
    s = lax.axis_index("s")
    pltpu.sync_copy(identh, idv)

    @pl.when(s == 0)
    def _():
        pltpu.sync_copy(z3.at[0], dshared.at[pl.ds(0, 40)])
        pltpu.sync_copy(z3.at[1], dshared.at[pl.ds(40, 40)])

    plsc.subcore_barrier()
    pbase = c * PPB + s * NCHUNK
    iota16 = lax.iota(jnp.int32, 16)

    def hist(buf, lo):
        for q in range(CHUNK // 16):
            v = idx_v[buf, 1, pl.ds(q * 16, 16)]
            vv = v - lo
            m = jnp.logical_and(vv >= 0, vv < HALF)
            vc = jnp.clip(vv, 0, HALF - 1)
            hi = lax.shift_right_logical(vc, 7)
            lo7 = lax.bitwise_and(vc, 127)
            cur = plsc.load_gather(ldeg, [iota16, hi, lo7], mask=m)
            plsc.store_scatter(ldeg, [iota16, hi, lo7], cur + 1.0, mask=m)

    def fire(t, buf):
        pltpu.async_copy(pairs.at[pbase + t], idx_v.at[buf], sems.at[buf])

    def drain(t, buf):
        pltpu.make_async_copy(pairs.at[pbase + t], idx_v.at[buf],
                              sems.at[buf]).wait()

    for half in range(2):
        lo = half * HALF
        pltpu.sync_copy(z3, ldeg)
        fire(0, 0)

        def chunk(tt, carry):
            t0 = tt * 2
            fire(t0 + 1, 1)
            drain(t0, 0)
            hist(0, lo)

            @pl.when(t0 + 2 < NCHUNK)
            def _():
                fire(t0 + 2, 0)

            drain(t0 + 1, 1)
            hist(1, lo)
            return carry

        lax.fori_loop(0, NCHUNK // 2, chunk, 0)
        for k2 in range(16):
            pltpu.sync_copy(ldeg.at[k2], dshared.at[idv.at[half]], add=True)
    plsc.subcore_barrier()

    @pl.when(s == 0)
    def _():
        pltpu.sync_copy(dshared, dout.at[pl.ds(c * 80, 80)])


@functools.lru_cache(maxsize=1)
def _sc_kernels():
    mesh = plsc.VectorSubcoreMesh(core_axis_name="c", subcore_axis_name="s")
    seg_scratch = [
        pltpu.VMEM((2, 2, CHUNK), jnp.int32),
        pltpu.VMEM((2, CHUNK, IN_DIM), jnp.float32),
        pltpu.SemaphoreType.DMA((2,)),
        pltpu.VMEM_SHARED((ACC_ROWS, IN_DIM), jnp.float32),
        pltpu.SemaphoreType.DMA,
    ]
    seg1 = pl.kernel(
        _seg_body(1),
        out_type=jax.ShapeDtypeStruct((NC * N_NODES, IN_DIM), jnp.float32),
        mesh=mesh,
        scratch_types=list(seg_scratch),
    )
    seg2 = pl.kernel(
        _seg_body(2),
        out_type=jax.ShapeDtypeStruct((2 * NC * N_NODES, IN_DIM), jnp.float32),
        mesh=mesh,
        scratch_types=list(seg_scratch),
    )
    deg = pl.kernel(
        _deg_body,
        out_type=jax.ShapeDtypeStruct((NC * 80, 128), jnp.float32),
        compiler_params=pltpu.CompilerParams(needs_layout_passes=False),
        mesh=mesh,
        scratch_types=[
            pltpu.VMEM((2, 2, CHUNK), jnp.int32),
            pltpu.VMEM((16, 40, 128), jnp.float32),
            pltpu.VMEM((2, 40), jnp.int32),
            pltpu.SemaphoreType.DMA((2,)),
            pltpu.VMEM_SHARED((80, 128), jnp.float32),
            pltpu.SemaphoreType.DMA,
        ],
    )
    return seg1, seg2, deg


R = 400
NB = N_NODES // R


def _tc1_body(feats_ref, s1_ref, deg_ref, ws_ref, wn_ref, b_ref, y_ref):
    d = jnp.maximum(deg_ref[...], 1.0)
    hn = s1_ref[...] / d
    y = (jnp.dot(feats_ref[...], ws_ref[0], preferred_element_type=jnp.float32)
         + jnp.dot(hn, wn_ref[0], preferred_element_type=jnp.float32)
         + b_ref[0, 0:1, :])
    y_ref[...] = jnp.maximum(y, 0.0)


def _tc1(feats, s1, deg, ws, wn, b):
    return pl.pallas_call(
        _tc1_body,
        grid=(NB, 4),
        in_specs=[
            pl.BlockSpec((R, IN_DIM), lambda i, j: (i, 0)),
            pl.BlockSpec((R, IN_DIM), lambda i, j: ((j // 2) * NB + i, 0)),
            pl.BlockSpec((R, 1), lambda i, j: ((j // 2) * NB + i, 0)),
            pl.BlockSpec((1, IN_DIM, IN_DIM), lambda i, j: (j // 2, 0, j % 2)),
            pl.BlockSpec((1, IN_DIM, IN_DIM), lambda i, j: (j // 2, 0, j % 2)),
            pl.BlockSpec((1, 8, IN_DIM), lambda i, j: (j // 2, 0, j % 2)),
        ],
        out_specs=pl.BlockSpec((R, IN_DIM), lambda i, j: (j * NB + i, 0)),
        out_shape=jax.ShapeDtypeStruct((2 * NC * N_NODES, IN_DIM), jnp.float32),
    )(feats, s1, deg, ws, wn, b)


def _tc2_body(yl_ref, yr_ref, s2l_ref, s2r_ref, deg_ref,
              ws_ref, wn_ref, b_ref, wl_ref, out_ref, acc_ref):
    bidx = pl.program_id(0)
    i = pl.program_id(1)
    h1 = jnp.concatenate([yl_ref[...], yr_ref[...]], axis=1)
    d = jnp.maximum(deg_ref[...], 1.0)
    hn = jnp.concatenate([s2l_ref[...], s2r_ref[...]], axis=1) / d
    h2 = (jnp.dot(h1, ws_ref[0], preferred_element_type=jnp.float32)
          + jnp.dot(hn, wn_ref[0], preferred_element_type=jnp.float32)
          + b_ref[0, 0:1, :])
    h2 = jnp.maximum(h2, 0.0)
    colsum = jnp.sum(h2, axis=0, keepdims=True)

    @pl.when(i == 0)
    def _():
        acc_ref[pl.ds(bidx, 1), :] = colsum

    @pl.when(i > 0)
    def _():
        acc_ref[pl.ds(bidx, 1), :] = acc_ref[pl.ds(bidx, 1), :] + colsum

    @pl.when(jnp.logical_and(bidx == 1, i == NB - 1))
    def _():
        inv = 1.0 / N_NODES
        ra = jnp.dot(acc_ref[0:1, :] * inv, wl_ref[0],
                     preferred_element_type=jnp.float32)
        ra = jnp.maximum(ra, 0.0)
        rb = jnp.dot(acc_ref[1:2, :] * inv, wl_ref[1],
                     preferred_element_type=jnp.float32)
        out_ref[...] = jax.nn.sigmoid((ra + rb) * 0.5)


def _tc2(y, s2, deg, ws, wn, b, wl):
    return pl.pallas_call(
        _tc2_body,
        grid=(2, NB),
        in_specs=[
            pl.BlockSpec((R, IN_DIM), lambda bi, i: ((bi * 2) * NB + i, 0)),
            pl.BlockSpec((R, IN_DIM), lambda bi, i: ((bi * 2 + 1) * NB + i, 0)),
            pl.BlockSpec((R, IN_DIM), lambda bi, i: ((bi * 2) * NB + i, 0)),
            pl.BlockSpec((R, IN_DIM), lambda bi, i: ((bi * 2 + 1) * NB + i, 0)),
            pl.BlockSpec((R, 1), lambda bi, i: (bi * NB + i, 0)),
            pl.BlockSpec((1, HID, HID), lambda bi, i: (bi, 0, 0)),
            pl.BlockSpec((1, HID, HID), lambda bi, i: (bi, 0, 0)),
            pl.BlockSpec((1, 8, HID), lambda bi, i: (bi, 0, 0)),
            pl.BlockSpec((2, HID, OUT), lambda bi, i: (0, 0, 0)),
        ],
        out_specs=pl.BlockSpec((1, OUT), lambda bi, i: (0, 0)),
        out_shape=jax.ShapeDtypeStruct((1, OUT), jnp.float32),
        scratch_shapes=[pltpu.VMEM((8, HID), jnp.float32)],
    )(y, y, s2, s2, deg, ws, wn, b, wl)


def kernel(feats, edge_index1, edge_index2,
           W1_self_a, W1_neigh_a, b1_a, W2_self_a, W2_neigh_a, b2_a, Wl_a,
           W1_self_b, W1_neigh_b, b1_b, W2_self_b, W2_neigh_b, b2_b, Wl_b):
    e1 = edge_index1.astype(jnp.int32)
    e2 = edge_index2.astype(jnp.int32)

    def pack(src, dst, srcoff):
        s = jnp.pad((src + srcoff).reshape(NS, EPT), ((0, 0), (0, EPAD)))
        d = jnp.pad(dst.reshape(NS, EPT), ((0, 0), (0, EPAD)),
                    constant_values=N_NODES)
        s = s.reshape(NS, NCHUNK, 1, CHUNK)
        d = d.reshape(NS, NCHUNK, 1, CHUNK)
        return jnp.concatenate([s, d], axis=2).reshape(PPB, 2, CHUNK)

    pairs1 = jnp.concatenate([pack(e1[0], e1[1], 0),
                              pack(e2[0], e2[1], 0)])
    pairs2 = jnp.concatenate([pack(e1[0], e1[1], 0),
                              pack(e1[0], e1[1], N_NODES),
                              pack(e2[0], e2[1], 2 * N_NODES),
                              pack(e2[0], e2[1], 3 * N_NODES)])
    z128 = jnp.zeros((RPT, IN_DIM), jnp.float32)
    z3 = jnp.zeros((16, 40, 128), jnp.float32)
    ident2 = jnp.arange(2 * 40, dtype=jnp.int32).reshape(2, 40)

    sc_seg1, sc_seg2, sc_deg = _sc_kernels()
    s1 = sc_seg1(feats, pairs1, z128)
    deg_raw = sc_deg(pairs1, z3, ident2)
    deg = deg_raw.reshape(NC, 80 * 128)[:, :N_NODES].reshape(NC * N_NODES, 1)

    ws1 = jnp.stack([W1_self_a, W1_self_b])
    wn1 = jnp.stack([W1_neigh_a, W1_neigh_b])
    b1 = jnp.tile(jnp.stack([b1_a, b1_b])[:, None, :], (1, 8, 1))
    y = _tc1(feats, s1, deg, ws1, wn1, b1)

    s2 = sc_seg2(y, pairs2, z128)

    ws2 = jnp.stack([W2_self_a, W2_self_b])
    wn2 = jnp.stack([W2_neigh_a, W2_neigh_b])
    b2 = jnp.tile(jnp.stack([b2_a, b2_b])[:, None, :], (1, 8, 1))
    wl = jnp.stack([Wl_a, Wl_b])
    return _tc2(y, s2, deg, ws2, wn2, b2, wl)
